# Initial kernel scaffold; baseline (speedup 1.0000x reference)
import jax
import jax.numpy as jnp
from jax.experimental import pallas as pl


def kernel(x, weights, top_weights, top_experts, w1, v1, w2):
    raise NotImplementedError("write your pallas kernel here")


# speedup vs baseline: 1.4101x; 1.4101x over previous
import functools

import jax
import jax.numpy as jnp
from jax import lax
from jax.experimental import pallas as pl
from jax.experimental.pallas import tpu as pltpu
from jax.experimental.pallas import tpu_sc as plsc

_E = 8
_FFN = 4096
_H = 1024
_K = 2

_TILE = 256
_FCH = 2048
_NF = _FFN // _FCH

_NW = 32


def _sc_gather(table, idx):
    b = idx.shape[0]
    d = table.shape[1]
    per_w = b // _NW
    max_rows = (128 * 1024) // (d * table.dtype.itemsize)
    chunk = (min(per_w, max_rows) // 8) * 8
    while per_w % chunk != 0:
        chunk -= 8
    nchunk = per_w // chunk
    mesh = plsc.VectorSubcoreMesh(core_axis_name="c", subcore_axis_name="s")

    @functools.partial(
        pl.kernel,
        out_type=jax.ShapeDtypeStruct((b, d), table.dtype),
        mesh=mesh,
        scratch_types=[
            pltpu.VMEM((chunk,), jnp.int32),
            pltpu.VMEM((chunk, d), table.dtype),
            pltpu.SemaphoreType.DMA,
        ],
    )
    def k(table_hbm, idx_hbm, out_hbm, idx_v, rows_v, sem):
        wid = lax.axis_index("s") * 2 + lax.axis_index("c")

        @pl.loop(0, nchunk)
        def _(c):
            base = wid * per_w + c * chunk
            pltpu.sync_copy(idx_hbm.at[pl.ds(base, chunk)], idx_v)
            pltpu.async_copy(table_hbm.at[idx_v], rows_v, sem).wait()
            pltpu.sync_copy(rows_v, out_hbm.at[pl.ds(base, chunk)])

    return k(table, idx)


def _mlp_body(te_ref, xg_ref, w1_ref, v1_ref, w2_ref, pw_ref, y_ref, acc_ref):
    f = pl.program_id(1)

    @pl.when(f == 0)
    def _():
        acc_ref[...] = jnp.zeros_like(acc_ref)

    xb = xg_ref[...]
    gate = lax.dot_general(xb, w1_ref[0], (((1,), (1,)), ((), ())),
                           preferred_element_type=jnp.float32)
    up = lax.dot_general(xb, v1_ref[0], (((1,), (1,)), ((), ())),
                         preferred_element_type=jnp.float32)
    inter = (gate * jax.nn.sigmoid(gate) * up).astype(jnp.bfloat16)
    acc_ref[...] += lax.dot_general(inter, w2_ref[0], (((1,), (0,)), ((), ())),
                                    preferred_element_type=jnp.float32)

    @pl.when(f == _NF - 1)
    def _():
        y_ref[...] = acc_ref[...] * pw_ref[...]


def _grouped_mlp(xg, w1b, v1b, w2b, pw, te):
    np_rows = xg.shape[0]
    nt = np_rows // _TILE
    grid = (nt, _NF)
    return pl.pallas_call(
        _mlp_body,
        grid_spec=pltpu.PrefetchScalarGridSpec(
            num_scalar_prefetch=1,
            grid=grid,
            in_specs=[
                pl.BlockSpec((_TILE, _H), lambda i, f, te: (i, 0)),
                pl.BlockSpec((1, _FCH, _H), lambda i, f, te: (te[i], f, 0)),
                pl.BlockSpec((1, _FCH, _H), lambda i, f, te: (te[i], f, 0)),
                pl.BlockSpec((1, _FCH, _H), lambda i, f, te: (te[i], f, 0)),
                pl.BlockSpec((_TILE, 1), lambda i, f, te: (i, 0)),
            ],
            out_specs=pl.BlockSpec((_TILE, _H), lambda i, f, te: (i, 0)),
            scratch_shapes=[pltpu.VMEM((_TILE, _H), jnp.float32)],
        ),
        out_shape=jax.ShapeDtypeStruct((np_rows, _H), jnp.float32),
        compiler_params=pltpu.CompilerParams(
            dimension_semantics=("parallel", "arbitrary")),
    )(te, xg, w1b, v1b, w2b, pw)


def _add_body(a_ref, b_ref, o_ref):
    o_ref[...] = a_ref[0] + b_ref[0]


def _add_halves(z):
    t = z.shape[1]
    blk = 512
    return pl.pallas_call(
        _add_body,
        grid=(t // blk,),
        in_specs=[
            pl.BlockSpec((1, blk, _H), lambda i: (0, i, 0)),
            pl.BlockSpec((1, blk, _H), lambda i: (1, i, 0)),
        ],
        out_specs=pl.BlockSpec((blk, _H), lambda i: (i, 0)),
        out_shape=jax.ShapeDtypeStruct((t, _H), jnp.float32),
    )(z, z)


def kernel(x, weights, top_weights, top_experts, w1, v1, w2):
    bsz, q_len, hidden = x.shape
    t = bsz * q_len
    n = t * _K
    np_rows = n + _E * _TILE
    nt = np_rows // _TILE

    xf = x.reshape(t, hidden)

    e_flat = top_experts.reshape(-1).astype(jnp.int32)
    w_flat = top_weights.reshape(-1).astype(jnp.float32)
    tok = jnp.arange(n, dtype=jnp.int32) // _K
    oh = (e_flat[:, None] == jnp.arange(_E, dtype=jnp.int32)[None, :])
    ranks = jnp.cumsum(oh.astype(jnp.int32), axis=0) - 1
    counts = jnp.sum(oh.astype(jnp.int32), axis=0)
    pc = ((counts + _TILE - 1) // _TILE) * _TILE
    cum_pc = jnp.cumsum(pc)
    pstart = cum_pc - pc
    rank = jnp.take_along_axis(ranks, e_flat[:, None], axis=1)[:, 0]
    slot = pstart[e_flat] + rank
    row_tok = jnp.zeros((np_rows,), jnp.int32).at[slot].set(tok)
    pair_w = jnp.zeros((np_rows, 1), jnp.float32).at[slot, 0].set(w_flat)
    tile_start = jnp.arange(nt, dtype=jnp.int32) * _TILE
    te = jnp.minimum(
        jnp.searchsorted(cum_pc, tile_start, side="right"), _E - 1
    ).astype(jnp.int32)
    p0 = slot[0::2]
    p1 = slot[1::2]

    xg = _sc_gather(xf, row_tok)

    w1b = w1.reshape(_E, _FFN, _H).astype(jnp.bfloat16)
    v1b = v1.reshape(_E, _FFN, _H).astype(jnp.bfloat16)
    w2b = w2.reshape(_E, _FFN, _H).astype(jnp.bfloat16)
    y = _grouped_mlp(xg.astype(jnp.bfloat16), w1b, v1b, w2b, pair_w, te)

    z = _sc_gather(y, jnp.concatenate([p0, p1]))
    out = _add_halves(z.reshape(2, t, _H))
    return out.reshape(bsz, q_len, hidden)


# speedup vs baseline: 1.8266x; 1.2954x over previous
import functools

import jax
import jax.numpy as jnp
from jax import lax
from jax.experimental import pallas as pl
from jax.experimental.pallas import tpu as pltpu
from jax.experimental.pallas import tpu_sc as plsc

_E = 8
_FFN = 4096
_H = 1024
_K = 2

_TILE = 256
_FCH = 2048
_NF = _FFN // _FCH

_NW = 32


def _sc_gather(table, idx):
    b = idx.shape[0]
    d = table.shape[1]
    per_w = b // _NW
    max_rows = (128 * 1024) // (d * table.dtype.itemsize)
    chunk = (min(per_w, max_rows) // 8) * 8
    while per_w % chunk != 0:
        chunk -= 8
    nchunk = per_w // chunk
    mesh = plsc.VectorSubcoreMesh(core_axis_name="c", subcore_axis_name="s")

    @functools.partial(
        pl.kernel,
        out_type=jax.ShapeDtypeStruct((b, d), table.dtype),
        mesh=mesh,
        scratch_types=[
            pltpu.VMEM((chunk,), jnp.int32),
            pltpu.VMEM((chunk, d), table.dtype),
            pltpu.SemaphoreType.DMA,
        ],
    )
    def k(table_hbm, idx_hbm, out_hbm, idx_v, rows_v, sem):
        wid = lax.axis_index("s") * 2 + lax.axis_index("c")

        @pl.loop(0, nchunk)
        def _(c):
            base = wid * per_w + c * chunk
            pltpu.sync_copy(idx_hbm.at[pl.ds(base, chunk)], idx_v)
            pltpu.async_copy(table_hbm.at[idx_v], rows_v, sem).wait()
            pltpu.sync_copy(rows_v, out_hbm.at[pl.ds(base, chunk)])

    return k(table, idx)


_DEFAULT = lax.Precision.DEFAULT


def _gateup_body(te_ref, xg_ref, w1_ref, v1_ref, inter_ref):
    xb = xg_ref[...]
    gate = lax.dot_general(xb, w1_ref[0], (((1,), (1,)), ((), ())),
                           preferred_element_type=jnp.float32,
                           precision=_DEFAULT)
    up = lax.dot_general(xb, v1_ref[0], (((1,), (1,)), ((), ())),
                         preferred_element_type=jnp.float32,
                         precision=_DEFAULT)
    inter_ref[...] = (gate * jax.nn.sigmoid(gate) * up).astype(jnp.bfloat16)


def _down_body(te_ref, inter_ref, w2_ref, pw_ref, y_ref):
    d = lax.dot_general(inter_ref[...], w2_ref[0], (((1,), (0,)), ((), ())),
                        preferred_element_type=jnp.float32,
                        precision=_DEFAULT)
    y_ref[...] = d * pw_ref[...]


def _grouped_mlp(xg, w1c, v1c, w2c, pw, te):
    np_rows = xg.shape[0]
    nt = np_rows // _TILE
    inter = pl.pallas_call(
        _gateup_body,
        grid_spec=pltpu.PrefetchScalarGridSpec(
            num_scalar_prefetch=1,
            grid=(_NF, nt),
            in_specs=[
                pl.BlockSpec((_TILE, _H), lambda f, i, te: (i, 0)),
                pl.BlockSpec((1, _FCH, _H), lambda f, i, te: (te[i], f, 0)),
                pl.BlockSpec((1, _FCH, _H), lambda f, i, te: (te[i], f, 0)),
            ],
            out_specs=pl.BlockSpec((_TILE, _FCH), lambda f, i, te: (i, f)),
        ),
        out_shape=jax.ShapeDtypeStruct((np_rows, _FFN), jnp.bfloat16),
        compiler_params=pltpu.CompilerParams(
            dimension_semantics=("arbitrary", "arbitrary")),
    )(te, xg, w1c, v1c)
    return pl.pallas_call(
        _down_body,
        grid_spec=pltpu.PrefetchScalarGridSpec(
            num_scalar_prefetch=1,
            grid=(nt,),
            in_specs=[
                pl.BlockSpec((_TILE, _FFN), lambda i, te: (i, 0)),
                pl.BlockSpec((1, _FFN, _H), lambda i, te: (te[i], 0, 0)),
                pl.BlockSpec((_TILE, 1), lambda i, te: (i, 0)),
            ],
            out_specs=pl.BlockSpec((_TILE, _H), lambda i, te: (i, 0)),
        ),
        out_shape=jax.ShapeDtypeStruct((np_rows, _H), jnp.float32),
        compiler_params=pltpu.CompilerParams(
            dimension_semantics=("arbitrary",)),
    )(te, inter, w2c, pw)


def _add_body(a_ref, b_ref, o_ref):
    o_ref[...] = a_ref[0] + b_ref[0]


def _add_halves(z):
    t = z.shape[1]
    blk = 512
    return pl.pallas_call(
        _add_body,
        grid=(t // blk,),
        in_specs=[
            pl.BlockSpec((1, blk, _H), lambda i: (0, i, 0)),
            pl.BlockSpec((1, blk, _H), lambda i: (1, i, 0)),
        ],
        out_specs=pl.BlockSpec((blk, _H), lambda i: (i, 0)),
        out_shape=jax.ShapeDtypeStruct((t, _H), jnp.float32),
    )(z, z)


def kernel(x, weights, top_weights, top_experts, w1, v1, w2):
    bsz, q_len, hidden = x.shape
    t = bsz * q_len
    n = t * _K
    np_rows = n + _E * _TILE
    nt = np_rows // _TILE

    xf = x.reshape(t, hidden)

    e_flat = top_experts.reshape(-1).astype(jnp.int32)
    w_flat = top_weights.reshape(-1).astype(jnp.float32)
    tok = jnp.arange(n, dtype=jnp.int32) // _K
    oh = (e_flat[:, None] == jnp.arange(_E, dtype=jnp.int32)[None, :])
    ranks = jnp.cumsum(oh.astype(jnp.int32), axis=0) - 1
    counts = jnp.sum(oh.astype(jnp.int32), axis=0)
    pc = ((counts + _TILE - 1) // _TILE) * _TILE
    cum_pc = jnp.cumsum(pc)
    pstart = cum_pc - pc
    rank = jnp.take_along_axis(ranks, e_flat[:, None], axis=1)[:, 0]
    slot = pstart[e_flat] + rank
    row_tok = jnp.zeros((np_rows,), jnp.int32).at[slot].set(tok)
    pair_w = jnp.zeros((np_rows, 1), jnp.float32).at[slot, 0].set(w_flat)
    tile_start = jnp.arange(nt, dtype=jnp.int32) * _TILE
    te = jnp.minimum(
        jnp.searchsorted(cum_pc, tile_start, side="right"), _E - 1
    ).astype(jnp.int32)
    p0 = slot[0::2]
    p1 = slot[1::2]

    xg = _sc_gather(xf, row_tok)

    w1c = w1.reshape(_E, _FFN, _H)
    v1c = v1.reshape(_E, _FFN, _H)
    w2c = w2.reshape(_E, _FFN, _H)
    y = _grouped_mlp(xg, w1c, v1c, w2c, pair_w, te)

    z = _sc_gather(y, jnp.concatenate([p0, p1]))
    out = _add_halves(z.reshape(2, t, _H))
    return out.reshape(bsz, q_len, hidden)
